# baseline (speedup 1.0000x reference)
import functools

import jax
import jax.numpy as jnp
from jax import lax
from jax.experimental import pallas as pl
from jax.experimental.pallas import tpu as pltpu
from jax.experimental.pallas import tpu_sc as plsc

INPUT_DIM = 2048
B, T = 4, 4096
ROWS = B * T
NC, NS, LANES = 2, 16, 16
NW = NC * NS
RPW = ROWS // NW
CHUNKS = INPUT_DIM // LANES


def _sc_body(x_hbm, mu_hbm, g_hbm, out_hbm, mu_v, g_v, idx_v, col_v, sem):
    c = lax.axis_index("c")
    s = lax.axis_index("s")
    wid = s * NC + c

    pltpu.sync_copy(mu_hbm, mu_v)
    pltpu.sync_copy(g_hbm, g_v)

    lanes = jnp.arange(LANES, dtype=jnp.int32)
    init_val = jnp.full((LANES,), -3.0e38, jnp.float32)
    init_idx = jnp.zeros((LANES,), jnp.int32)

    def body(i, carry):
        bv, bi = carry
        off = i * LANES
        y = mu_v[pl.ds(off, LANES)] + g_v[pl.ds(off, LANES)]
        flat = off + lanes
        take = y > bv
        return jnp.where(take, y, bv), jnp.where(take, flat, bi)

    bv, bi = lax.fori_loop(0, CHUNKS, body, (init_val, init_idx))
    best = jnp.float32(-3.0e38)
    idx = jnp.int32(2**30)
    for j in range(LANES):
        v = bv[j]
        fi = bi[j]
        take = (v > best) | ((v == best) & (fi < idx))
        best = jnp.where(take, v, best)
        idx = jnp.where(take, fi, idx)

    base = wid * RPW

    def ibody(j, _):
        idx_v[pl.ds(j * LANES, LANES)] = (base + j * LANES + lanes) * INPUT_DIM + idx
        return 0

    lax.fori_loop(0, RPW // LANES, ibody, 0)

    copies = [
        pltpu.async_copy(
            x_hbm.at[idx_v.at[pl.ds(cidx * 128, 128)]],
            col_v.at[pl.ds(cidx * 128, 128)],
            sem,
        )
        for cidx in range(RPW // 128)
    ]
    for cp in copies:
        cp.wait()
    pltpu.sync_copy(col_v, out_hbm.at[wid])


_sc_gather = functools.partial(
    pl.kernel,
    mesh=plsc.VectorSubcoreMesh(core_axis_name="c", subcore_axis_name="s"),
    out_type=jax.ShapeDtypeStruct((NW, RPW), jnp.float32),
    scratch_types=[
        pltpu.VMEM((INPUT_DIM,), jnp.float32),
        pltpu.VMEM((INPUT_DIM,), jnp.float32),
        pltpu.VMEM((RPW,), jnp.int32),
        pltpu.VMEM((RPW,), jnp.float32),
        pltpu.SemaphoreType.DMA,
    ],
)(_sc_body)


def kernel(x, mu):
    u = jax.random.uniform(jax.random.key(42), (INPUT_DIM,),
                           minval=1e-10, maxval=1.0)
    g = -jnp.log(-jnp.log(u))
    x1 = x.reshape(ROWS * INPUT_DIM)
    out = _sc_gather(x1, mu, g)
    return out.reshape(B, T, 1)


# speedup vs baseline: 4.2426x; 4.2426x over previous
import functools

import jax
import jax.numpy as jnp
from jax import lax
from jax.experimental import pallas as pl
from jax.experimental.pallas import tpu as pltpu
from jax.experimental.pallas import tpu_sc as plsc

INPUT_DIM = 2048
B, T = 4, 4096
ROWS = B * T
NC, NS, LANES = 2, 16, 16
NW = NC * NS
RPW = ROWS // NW
CHUNKS = INPUT_DIM // LANES


def _sc_body(x_hbm, mu_hbm, g_hbm, out_hbm, mu_v, g_v, blk_v, col_v):
    c = lax.axis_index("c")
    s = lax.axis_index("s")
    wid = s * NC + c

    pltpu.sync_copy(mu_hbm, mu_v)
    pltpu.sync_copy(g_hbm, g_v)

    lanes = jnp.arange(LANES, dtype=jnp.int32)
    init_val = jnp.full((LANES,), -3.0e38, jnp.float32)
    init_idx = jnp.zeros((LANES,), jnp.int32)

    def body(i, carry):
        bv, bi = carry
        off = i * LANES
        y = mu_v[pl.ds(off, LANES)] + g_v[pl.ds(off, LANES)]
        flat = off + lanes
        take = y > bv
        return jnp.where(take, y, bv), jnp.where(take, flat, bi)

    bv, bi = lax.fori_loop(0, CHUNKS, body, (init_val, init_idx))
    best = jnp.float32(-3.0e38)
    idx = jnp.int32(2**30)
    for j in range(LANES):
        v = bv[j]
        fi = bi[j]
        take = (v > best) | ((v == best) & (fi < idx))
        best = jnp.where(take, v, best)
        idx = jnp.where(take, fi, idx)

    col0 = pl.multiple_of((idx // 128) * 128, 128)
    colmod = idx - col0
    base = wid * RPW
    pltpu.sync_copy(x_hbm.at[pl.ds(base, RPW), pl.ds(col0, 128)], blk_v)

    cvec = jnp.zeros((LANES,), jnp.int32) + colmod

    def gbody(r, _):
        row_idx = r * LANES + lanes
        vals = plsc.load_gather(blk_v, [row_idx, cvec])
        col_v[pl.ds(r * LANES, LANES)] = vals
        return 0

    lax.fori_loop(0, RPW // LANES, gbody, 0)
    pltpu.sync_copy(col_v, out_hbm.at[wid])


_sc_gather = functools.partial(
    pl.kernel,
    mesh=plsc.VectorSubcoreMesh(core_axis_name="c", subcore_axis_name="s"),
    out_type=jax.ShapeDtypeStruct((NW, RPW), jnp.float32),
    scratch_types=[
        pltpu.VMEM((INPUT_DIM,), jnp.float32),
        pltpu.VMEM((INPUT_DIM,), jnp.float32),
        pltpu.VMEM((RPW, 128), jnp.float32),
        pltpu.VMEM((RPW,), jnp.float32),
    ],
    compiler_params=pltpu.CompilerParams(needs_layout_passes=False),
)(_sc_body)


def kernel(x, mu):
    u = jax.random.uniform(jax.random.key(42), (INPUT_DIM,),
                           minval=1e-10, maxval=1.0)
    g = -jnp.log(-jnp.log(u))
    x2 = x.reshape(ROWS, INPUT_DIM)
    out = _sc_gather(x2, mu, g)
    return out.reshape(B, T, 1)
